# baseline (speedup 1.0000x reference)
import functools

import jax
import jax.numpy as jnp
from jax import lax
from jax.experimental import pallas as pl
from jax.experimental.pallas import tpu as pltpu
from jax.experimental.pallas import tpu_sc as plsc

F = 26
D = 32
B = 16384
NC, NS, L = 2, 16, 16
NW = NC * NS
BPW = B // NW
CB = 64
NCHUNK = BPW // CB
R = CB * F
G = 128
NG = R // G
NV = R // L


def _body(x_hbm, off_hbm, table_hbm, bias_hbm, out_hbm,
          offs_t, x_v, idx_v, rows_v, bias_v, sem):
    wid = lax.axis_index("s") * NC + lax.axis_index("c")
    base = wid * BPW

    pltpu.sync_copy(off_hbm, offs_t)
    pltpu.sync_copy(bias_hbm, bias_v)

    def chunk(c, _):
        fb = (base + c * CB) * F

        pltpu.sync_copy(x_hbm.at[pl.ds(fb, R)], x_v)

        def comp(i, _):
            v = x_v[pl.ds(i * L, L)] + offs_t[pl.ds(i * L, L)]
            idx_v[i // (G // L), pl.ds((i % (G // L)) * L, L)] = v
            return ()
        lax.fori_loop(0, NV, comp, ())

        cps = [pltpu.async_copy(table_hbm.at[idx_v.at[j]],
                                rows_v.at[pl.ds(j * G, G)], sem)
               for j in range(NG)]
        for cp in cps:
            cp.wait()

        for f in range(F):
            blo = bias_v[f, pl.ds(0, L)]
            bhi = bias_v[f, pl.ds(L, L)]
            def badd(b, _, f=f, blo=blo, bhi=bhi):
                r = b * F + f
                rows_v[r, pl.ds(0, L)] = rows_v[r, pl.ds(0, L)] + blo
                rows_v[r, pl.ds(L, L)] = rows_v[r, pl.ds(L, L)] + bhi
                return ()
            lax.fori_loop(0, CB, badd, ())

        pltpu.sync_copy(rows_v, out_hbm.at[pl.ds(fb, R)])
        return ()

    lax.fori_loop(0, NCHUNK, chunk, ())


@functools.partial(jax.jit, static_argnames=())
def kernel(x, table, bias, offsets):
    mesh = plsc.VectorSubcoreMesh(core_axis_name="c", subcore_axis_name="s")
    k = pl.kernel(
        _body,
        out_type=jax.ShapeDtypeStruct((B * F, D), jnp.float32),
        mesh=mesh,
        compiler_params=pltpu.CompilerParams(use_tc_tiling_on_sc=False),
        scratch_types=[
            pltpu.VMEM((R,), jnp.int32),
            pltpu.VMEM((R,), jnp.int32),
            pltpu.VMEM((NG, G), jnp.int32),
            pltpu.VMEM((R, D), jnp.float32),
            pltpu.VMEM((F, D), jnp.float32),
            pltpu.SemaphoreType.DMA,
        ],
    )
    offs_tiled = jnp.tile(offsets, CB)
    out = k(x.reshape(-1), offs_tiled, table, bias)
    return out.reshape(B, F, D)
